# Initial kernel scaffold; baseline (speedup 1.0000x reference)
import functools

import jax
import jax.numpy as jnp
from jax import lax
from jax.experimental import pallas as pl
from jax.experimental.pallas import tpu as pltpu
from jax.experimental.pallas import tpu_sc as plsc


def _gather_rows(flat, ids, num_rows, dm):
    info = plsc.get_sparse_core_info()
    nw = info.num_cores * info.num_subcores
    nc = info.num_cores

    rows_per_w = num_rows // nw
    chunk = 64
    while rows_per_w % chunk != 0:
        chunk //= 2
    n_chunks = rows_per_w // chunk

    ids3 = ids.reshape(nw, n_chunks, chunk)

    mesh = plsc.VectorSubcoreMesh(core_axis_name="c", subcore_axis_name="s")

    @functools.partial(
        pl.kernel,
        out_type=jax.ShapeDtypeStruct((num_rows, dm), jnp.float32),
        mesh=mesh,
        scratch_types=[
            pltpu.VMEM((n_chunks, chunk), jnp.int32),
            pltpu.VMEM((2, chunk, dm), jnp.float32),
            pltpu.SemaphoreType.DMA,
            pltpu.SemaphoreType.DMA,
            pltpu.SemaphoreType.DMA,
            pltpu.SemaphoreType.DMA,
        ],
    )
    def body(x_hbm, ids_hbm, out_hbm, idx_v, rows_v, g0, g1, w0, w1):
        wid = lax.axis_index("s") * nc + lax.axis_index("c")
        base = wid * rows_per_w

        pltpu.sync_copy(ids_hbm.at[wid], idx_v)

        gsems = (g0, g1)
        wsems = (w0, w1)
        gathers = [None, None]
        writes = [None, None]
        for c in range(n_chunks):
            b = c % 2
            if writes[b] is not None:
                writes[b].wait()
            gathers[b] = pltpu.async_copy(
                x_hbm.at[idx_v.at[c]], rows_v.at[b], gsems[b]
            )
            gathers[b].wait()
            writes[b] = pltpu.async_copy(
                rows_v.at[b], out_hbm.at[pl.ds(base + c * chunk, chunk)], wsems[b]
            )
        for w in writes:
            if w is not None:
                w.wait()

    return body(flat, ids3)


def kernel(x, ids_to_save, ids_unused):
    batch, _, dm = x.shape
    flat = x.reshape(-1, dm)
    ids = ids_to_save.astype(jnp.int32)
    num_rows = ids.shape[0]
    out = _gather_rows(flat, ids, num_rows, dm)
    return out.reshape(batch, -1, dm)


# speedup vs baseline: 1.5426x; 1.5426x over previous
import functools

import jax
import jax.numpy as jnp
from jax import lax
from jax.experimental import pallas as pl
from jax.experimental.pallas import tpu as pltpu
from jax.experimental.pallas import tpu_sc as plsc


def _gather_rows(flat, ids, num_rows, dm):
    info = plsc.get_sparse_core_info()
    nw = info.num_cores * info.num_subcores
    nc = info.num_cores

    rows_per_w = num_rows // nw
    chunk = 32
    while rows_per_w % chunk != 0:
        chunk //= 2
    n_chunks = rows_per_w // chunk

    ids3 = ids.reshape(nw, n_chunks, chunk)

    mesh = plsc.VectorSubcoreMesh(core_axis_name="c", subcore_axis_name="s")

    @functools.partial(
        pl.kernel,
        out_type=jax.ShapeDtypeStruct((num_rows, dm), jnp.float32),
        mesh=mesh,
        scratch_types=[
            pltpu.VMEM((n_chunks, chunk), jnp.int32),
            pltpu.VMEM((2, chunk, dm), jnp.float32),
            pltpu.SemaphoreType.DMA,
            pltpu.SemaphoreType.DMA,
            pltpu.SemaphoreType.DMA,
            pltpu.SemaphoreType.DMA,
        ],
    )
    def body(x_hbm, ids_hbm, out_hbm, idx_v, rows_v, g0, g1, w0, w1):
        wid = lax.axis_index("s") * nc + lax.axis_index("c")
        base = wid * rows_per_w

        pltpu.sync_copy(ids_hbm.at[wid], idx_v)

        gsems = (g0, g1)
        wsems = (w0, w1)
        gathers = [None, None]
        writes = [None, None]
        for c in range(n_chunks):
            b = c % 2
            if writes[b] is not None:
                writes[b].wait()
            gathers[b] = pltpu.async_copy(
                x_hbm.at[idx_v.at[c]], rows_v.at[b], gsems[b]
            )
            gathers[b].wait()
            writes[b] = pltpu.async_copy(
                rows_v.at[b], out_hbm.at[pl.ds(base + c * chunk, chunk)], wsems[b]
            )
        for w in writes:
            if w is not None:
                w.wait()

    return body(flat, ids3)


def kernel(x, ids_to_save, ids_unused):
    batch, _, dm = x.shape
    flat = x.reshape(-1, dm)
    ids = ids_to_save.astype(jnp.int32)
    num_rows = ids.shape[0]
    out = _gather_rows(flat, ids, num_rows, dm)
    return out.reshape(batch, -1, dm)


# speedup vs baseline: 1.6627x; 1.0779x over previous
import functools

import jax
import jax.numpy as jnp
from jax import lax
from jax.experimental import pallas as pl
from jax.experimental.pallas import tpu as pltpu
from jax.experimental.pallas import tpu_sc as plsc


def _gather_rows(flat, ids, num_rows, dm):
    info = plsc.get_sparse_core_info()
    nw = info.num_cores * info.num_subcores
    nc = info.num_cores

    rows_per_w = num_rows // nw
    chunk = 32
    while rows_per_w % chunk != 0:
        chunk //= 2
    n_chunks = rows_per_w // chunk

    ids3 = ids.reshape(nw, n_chunks, chunk)

    mesh = plsc.VectorSubcoreMesh(core_axis_name="c", subcore_axis_name="s")

    @functools.partial(
        pl.kernel,
        out_type=jax.ShapeDtypeStruct((num_rows, dm), jnp.float32),
        mesh=mesh,
        scratch_types=[
            pltpu.VMEM((n_chunks, chunk), jnp.int32),
            pltpu.VMEM((3, chunk, dm), jnp.float32),
            pltpu.SemaphoreType.DMA,
            pltpu.SemaphoreType.DMA,
            pltpu.SemaphoreType.DMA,
            pltpu.SemaphoreType.DMA,
            pltpu.SemaphoreType.DMA,
            pltpu.SemaphoreType.DMA,
        ],
    )
    def body(x_hbm, ids_hbm, out_hbm, idx_v, rows_v, g0, g1, g2, w0, w1, w2):
        wid = lax.axis_index("s") * nc + lax.axis_index("c")
        base = wid * rows_per_w

        pltpu.sync_copy(ids_hbm.at[wid], idx_v)

        gsems = (g0, g1, g2)
        wsems = (w0, w1, w2)
        nbuf = 3
        gathers = [None] * nbuf
        writes = [None] * nbuf

        def gather(c, b):
            return pltpu.async_copy(x_hbm.at[idx_v.at[c]], rows_v.at[b], gsems[b])

        def write(c, b):
            return pltpu.async_copy(
                rows_v.at[b], out_hbm.at[pl.ds(base + c * chunk, chunk)], wsems[b]
            )

        for c in range(n_chunks):
            b = c % nbuf
            if writes[b] is not None:
                writes[b].wait()
            gathers[b] = gather(c, b)
            p = c - (nbuf - 1)
            if p >= 0:
                pb = p % nbuf
                gathers[pb].wait()
                writes[pb] = write(p, pb)
        for p in range(max(0, n_chunks - (nbuf - 1)), n_chunks):
            pb = p % nbuf
            gathers[pb].wait()
            writes[pb] = write(p, pb)
        for w in writes:
            if w is not None:
                w.wait()

    return body(flat, ids3)


def kernel(x, ids_to_save, ids_unused):
    batch, _, dm = x.shape
    flat = x.reshape(-1, dm)
    ids = ids_to_save.astype(jnp.int32)
    num_rows = ids.shape[0]
    out = _gather_rows(flat, ids, num_rows, dm)
    return out.reshape(batch, -1, dm)


# speedup vs baseline: 1.6679x; 1.0031x over previous
import functools

import jax
import jax.numpy as jnp
from jax import lax
from jax.experimental import pallas as pl
from jax.experimental.pallas import tpu as pltpu
from jax.experimental.pallas import tpu_sc as plsc


def _gather_rows(flat, ids, num_rows, dm):
    info = plsc.get_sparse_core_info()
    nw = info.num_cores * info.num_subcores
    nc = info.num_cores

    rows_per_w = num_rows // nw
    chunk = 16
    while rows_per_w % chunk != 0:
        chunk //= 2
    n_chunks = rows_per_w // chunk
    nbuf = min(6, n_chunks)

    ids3 = ids.reshape(nw, n_chunks, chunk)

    mesh = plsc.VectorSubcoreMesh(core_axis_name="c", subcore_axis_name="s")

    @functools.partial(
        pl.kernel,
        out_type=jax.ShapeDtypeStruct((num_rows, dm), jnp.float32),
        mesh=mesh,
        scratch_types=[
            pltpu.VMEM((n_chunks, chunk), jnp.int32),
            pltpu.VMEM((nbuf, chunk, dm), jnp.float32),
            [pltpu.SemaphoreType.DMA] * nbuf,
            [pltpu.SemaphoreType.DMA] * nbuf,
        ],
    )
    def body(x_hbm, ids_hbm, out_hbm, idx_v, rows_v, gsems, wsems):
        wid = lax.axis_index("s") * nc + lax.axis_index("c")
        base = wid * rows_per_w

        pltpu.sync_copy(ids_hbm.at[wid], idx_v)

        gathers = [None] * nbuf
        writes = [None] * nbuf

        def gather(c, b):
            return pltpu.async_copy(x_hbm.at[idx_v.at[c]], rows_v.at[b], gsems[b])

        def write(c, b):
            return pltpu.async_copy(
                rows_v.at[b], out_hbm.at[pl.ds(base + c * chunk, chunk)], wsems[b]
            )

        for c in range(n_chunks):
            b = c % nbuf
            if writes[b] is not None:
                writes[b].wait()
            gathers[b] = gather(c, b)
            p = c - (nbuf - 1)
            if p >= 0:
                pb = p % nbuf
                gathers[pb].wait()
                writes[pb] = write(p, pb)
        for p in range(max(0, n_chunks - (nbuf - 1)), n_chunks):
            pb = p % nbuf
            gathers[pb].wait()
            writes[pb] = write(p, pb)
        for w in writes:
            if w is not None:
                w.wait()

    return body(flat, ids3)


def kernel(x, ids_to_save, ids_unused):
    batch, _, dm = x.shape
    flat = x.reshape(-1, dm)
    ids = ids_to_save.astype(jnp.int32)
    num_rows = ids.shape[0]
    out = _gather_rows(flat, ids, num_rows, dm)
    return out.reshape(batch, -1, dm)
